# baseline (speedup 1.0000x reference)
import functools

import jax
import jax.numpy as jnp
from jax import lax
from jax.experimental import pallas as pl
from jax.experimental.pallas import tpu as pltpu
from jax.experimental.pallas import tpu_sc as plsc

_SELU_SCALE = 1.0507009873554805
_SELU_ALPHA = 1.6732632423543772

_NC, _NS = 2, 16
_NW = _NC * _NS
_K = 64


def _num_chunks(E):
    nch = -(-E // (_NW * _K))
    return -(-nch // 4) * 4


@functools.lru_cache(maxsize=None)
def _make_sc_scatter(N, D, E):
    NCH = _num_chunks(E)
    NPAIR = NCH // 2
    assert D % 16 == 0 and _K % 16 == 0
    ROWS_PER_TILE = (N // _NS) // 8 * 8
    REM_ROWS = N - _NS * ROWS_PER_TILE
    assert REM_ROWS % 8 == 0 and REM_ROWS <= ROWS_PER_TILE

    mesh = plsc.VectorSubcoreMesh(core_axis_name="c", subcore_axis_name="s")

    @functools.partial(
        pl.kernel,
        mesh=mesh,
        out_type=jax.ShapeDtypeStruct((_NC, N, D), jnp.float32),
        scratch_types=[
            pltpu.VMEM((8, _K), jnp.int32),
            pltpu.VMEM((8, _K), jnp.int32),
            pltpu.VMEM((2, _K), jnp.float32),
            pltpu.VMEM((2, _K), jnp.float32),
            pltpu.VMEM((_K, 2 * D), jnp.float32),
            pltpu.VMEM((_K, 2 * D), jnp.float32),
            pltpu.VMEM_SHARED((N, D), jnp.float32),
            pltpu.SemaphoreType.DMA,
            pltpu.SemaphoreType.DMA,
            pltpu.SemaphoreType.DMA,
            pltpu.SemaphoreType.DMA,
            pltpu.SemaphoreType.DMA,
            pltpu.SemaphoreType.DMA,
            pltpu.SemaphoreType.DMA,
            pltpu.SemaphoreType.DMA,
        ],
    )
    def sc_scatter(eidx_hbm, ew_hbm, feat_hbm, zeros_hbm, out_hbm,
                   slot0, slot1, wslot0, wslot1, rows_a, rows_b, agg,
                   sem_e0, sem_e1, sem_w0, sem_w1,
                   sem_ga, sem_gb, sem_sa, sem_sb):
        cid = lax.axis_index("c")
        sid = lax.axis_index("s")
        wid = sid * _NC + cid
        slots = (slot0, slot1)
        wslots = (wslot0, wslot1)
        sems_e = (sem_e0, sem_e1)
        sems_w = (sem_w0, sem_w1)

        def edata_pair(p):
            return eidx_hbm.at[wid, p]

        def wdata_pair(p):
            return ew_hbm.at[wid, p]

        pltpu.async_copy(edata_pair(0), slot0, sem_e0)
        pltpu.async_copy(edata_pair(1), slot1, sem_e1)
        pltpu.async_copy(wdata_pair(0), wslot0, sem_w0)
        pltpu.async_copy(wdata_pair(1), wslot1, sem_w1)

        base_row = sid * ROWS_PER_TILE
        pltpu.sync_copy(zeros_hbm.at[pl.ds(0, ROWS_PER_TILE)],
                        agg.at[pl.ds(base_row, ROWS_PER_TILE)])

        @pl.when(sid == _NS - 1)
        def _zero_rem():
            pltpu.sync_copy(zeros_hbm.at[pl.ds(0, REM_ROWS)],
                            agg.at[pl.ds(_NS * ROWS_PER_TILE, REM_ROWS)])

        pltpu.make_async_copy(edata_pair(0), slot0, sem_e0).wait()
        pltpu.async_copy(feat_hbm.at[slot0.at[0]], rows_a, sem_ga)
        pltpu.async_copy(feat_hbm.at[slot0.at[2]], rows_b, sem_gb)

        plsc.subcore_barrier()

        def scale(rows, wslot, wrow):
            def scale_body(g, carry):
                wv = wslot[wrow, pl.ds(g * 16, 16)]
                for e in range(16):
                    w = wv[e]
                    r0 = g * 16 + e
                    for j in range(D // 16):
                        rows[r0, pl.ds(j * 16, 16)] = (
                            rows[r0, pl.ds(j * 16, 16)] * w)
                return carry
            lax.fori_loop(0, _K // 16, scale_body, 0)

        def body(q, carry):
            for half in range(2):
                slot, wslot = slots[half], wslots[half]
                sem_e, sem_w = sems_e[half], sems_w[half]
                other = slots[1 - half]
                sem_eo = sems_e[1 - half]
                p = 2 * q + half
                pltpu.make_async_copy(feat_hbm.at[slot.at[0]], rows_a,
                                      sem_ga).wait()
                pltpu.make_async_copy(wdata_pair(p), wslot, sem_w).wait()
                pass
                pltpu.make_async_copy(feat_hbm.at[slot.at[2]], rows_b,
                                      sem_gb).wait()
                pass
                @pl.when(p < NPAIR - 1)
                def _advance():
                    pltpu.make_async_copy(edata_pair(p + 1), other,
                                          sem_eo).wait()
                    pltpu.async_copy(feat_hbm.at[other.at[0]], rows_a, sem_ga)
                    pltpu.async_copy(feat_hbm.at[other.at[2]], rows_b, sem_gb)

                    @pl.when(p < NPAIR - 2)
                    def _prefetch():
                        pltpu.async_copy(edata_pair(p + 2), slot, sem_e)
                        pltpu.async_copy(wdata_pair(p + 2), wslot, sem_w)

            return carry

        lax.fori_loop(0, NPAIR // 2, body, 0)

        plsc.subcore_barrier()
        pltpu.sync_copy(agg.at[pl.ds(base_row, ROWS_PER_TILE)],
                        out_hbm.at[cid, pl.ds(base_row, ROWS_PER_TILE)])

        @pl.when(sid == _NS - 1)
        def _copy_rem():
            pltpu.sync_copy(
                agg.at[pl.ds(_NS * ROWS_PER_TILE, REM_ROWS)],
                out_hbm.at[cid, pl.ds(_NS * ROWS_PER_TILE, REM_ROWS)])

    return sc_scatter


def _tc_body(x_ref, bp_ref, k_ref, bias_ref, skip_ref, o_ref):
    kmat = k_ref[...]
    k2 = kmat * skip_ref[...]
    bsum = bp_ref[0] + bp_ref[1]
    acc = jnp.dot(x_ref[...], k2, preferred_element_type=jnp.float32,
                  precision=lax.Precision.HIGHEST)
    acc = acc + jnp.dot(bsum, kmat, preferred_element_type=jnp.float32,
                        precision=lax.Precision.HIGHEST)
    acc = acc + bias_ref[...]
    pos = acc > 0.0
    safe = jnp.where(pos, 0.0, acc)
    o_ref[...] = jnp.where(
        pos, _SELU_SCALE * acc,
        (_SELU_SCALE * _SELU_ALPHA) * (jnp.exp(safe) - 1.0))


def _tc_fused(features, bp, kmat, bias, skip):
    N, D = features.shape
    C = kmat.shape[1]
    BM = 1000
    grid = (N // BM,)
    return pl.pallas_call(
        _tc_body,
        grid=grid,
        in_specs=[
            pl.BlockSpec((BM, D), lambda i: (i, 0)),
            pl.BlockSpec((2, BM, C), lambda i: (0, i, 0)),
            pl.BlockSpec((D, C), lambda i: (0, 0)),
            pl.BlockSpec((1, C), lambda i: (0, 0)),
            pl.BlockSpec((1, C), lambda i: (0, 0)),
        ],
        out_specs=pl.BlockSpec((BM, C), lambda i: (i, 0)),
        out_shape=jax.ShapeDtypeStruct((N, C), jnp.float32),
    )(features, bp, kmat, bias, skip)


def kernel(features, edge_index, edge_weight, kernel, bias, skip_weight):
    N, D = features.shape
    C = kernel.shape[1]
    E = edge_weight.shape[0]
    NCH = _num_chunks(E)
    E2 = _NW * NCH * _K
    dst = edge_index[0]
    src = edge_index[1]
    pad = E2 - E
    src_p = jnp.concatenate(
        [src, jnp.zeros((pad,), jnp.int32)]).reshape(_NW, NCH // 2, 2, _K)
    dst_p = jnp.concatenate(
        [dst, jnp.zeros((pad,), jnp.int32)]).reshape(_NW, NCH // 2, 2, _K)
    w_p = jnp.concatenate(
        [edge_weight, jnp.zeros((pad,), jnp.float32)]
    ).reshape(_NW, NCH // 2, 2, _K)
    eidx = jnp.pad(
        jnp.stack([src_p[:, :, 0], dst_p[:, :, 0],
                   src_p[:, :, 1], dst_p[:, :, 1]], axis=2),
        ((0, 0), (0, 0), (0, 4), (0, 0)))
    zrows = (N // _NS) // 8 * 8
    zeros = jnp.zeros((zrows, D), jnp.float32)
    bp = _make_sc_scatter(N, D, E)(eidx, w_p, jnp.concatenate([features, features], axis=1), zeros)
    return _tc_fused(features, bp, kernel,
                     bias.reshape(1, C), skip_weight.reshape(1, C))


# speedup vs baseline: 1.4406x; 1.4406x over previous
import functools

import jax
import jax.numpy as jnp
from jax import lax
from jax.experimental import pallas as pl
from jax.experimental.pallas import tpu as pltpu
from jax.experimental.pallas import tpu_sc as plsc

_SELU_SCALE = 1.0507009873554805
_SELU_ALPHA = 1.6732632423543772

_NC, _NS = 2, 16
_K = 128
_SPLIT = (120, 40)


def _chunk_split(E):
    c0, c1 = _SPLIT
    total = c0 + c1
    need = -(-E // (_NS * _K))
    scale = max(1, -(-need // total))
    c0, c1 = c0 * scale, c1 * scale
    assert c0 % 4 == 0 and c1 % 4 == 0
    assert _NS * (c0 + c1) * _K >= E
    return c0, c1


@functools.lru_cache(maxsize=None)
def _make_sc_scatter(N, D, E):
    C0, C1 = _chunk_split(E)
    TP = _NS * (C0 + C1) // 2
    assert D % 16 == 0
    ROWS_PER_TILE = (N // _NS) // 8 * 8
    REM_ROWS = N - _NS * ROWS_PER_TILE
    assert REM_ROWS % 8 == 0 and REM_ROWS <= ROWS_PER_TILE

    mesh = plsc.VectorSubcoreMesh(core_axis_name="c", subcore_axis_name="s")

    @functools.partial(
        pl.kernel,
        mesh=mesh,
        out_type=jax.ShapeDtypeStruct((_NC, N, D), jnp.float32),
        scratch_types=[
            pltpu.VMEM((8, _K), jnp.int32),
            pltpu.VMEM((8, _K), jnp.int32),
            pltpu.VMEM((2, _K), jnp.float32),
            pltpu.VMEM((2, _K), jnp.float32),
            pltpu.VMEM((_K, D), jnp.float32),
            pltpu.VMEM((_K, D), jnp.float32),
            pltpu.VMEM_SHARED((N, D), jnp.float32),
            pltpu.SemaphoreType.DMA,
            pltpu.SemaphoreType.DMA,
            pltpu.SemaphoreType.DMA,
            pltpu.SemaphoreType.DMA,
            pltpu.SemaphoreType.DMA,
            pltpu.SemaphoreType.DMA,
            pltpu.SemaphoreType.DMA,
            pltpu.SemaphoreType.DMA,
        ],
    )
    def sc_scatter(eidx_hbm, ew_hbm, feat_hbm, zeros_hbm, out_hbm,
                   slot0, slot1, wslot0, wslot1, rows_a, rows_b, agg,
                   sem_e0, sem_e1, sem_w0, sem_w1,
                   sem_ga, sem_gb, sem_sa, sem_sb):
        cid = lax.axis_index("c")
        sid = lax.axis_index("s")
        slots = (slot0, slot1)
        wslots = (wslot0, wslot1)
        sems_e = (sem_e0, sem_e1)
        sems_w = (sem_w0, sem_w1)

        pair_base = jnp.where(cid == 0, sid * (C0 // 2),
                              _NS * (C0 // 2) + sid * (C1 // 2))
        my_npair = jnp.where(cid == 0, C0 // 2, C1 // 2)

        def edata_pair(p):
            return eidx_hbm.at[pair_base + p]

        def wdata_pair(p):
            return ew_hbm.at[pair_base + p]

        pltpu.async_copy(edata_pair(0), slot0, sem_e0)
        pltpu.async_copy(edata_pair(1), slot1, sem_e1)
        pltpu.async_copy(wdata_pair(0), wslot0, sem_w0)
        pltpu.async_copy(wdata_pair(1), wslot1, sem_w1)

        base_row = sid * ROWS_PER_TILE
        pltpu.sync_copy(zeros_hbm.at[pl.ds(0, ROWS_PER_TILE)],
                        agg.at[pl.ds(base_row, ROWS_PER_TILE)])

        @pl.when(sid == _NS - 1)
        def _zero_rem():
            pltpu.sync_copy(zeros_hbm.at[pl.ds(0, REM_ROWS)],
                            agg.at[pl.ds(_NS * ROWS_PER_TILE, REM_ROWS)])

        pltpu.make_async_copy(edata_pair(0), slot0, sem_e0).wait()
        pltpu.async_copy(feat_hbm.at[slot0.at[0]], rows_a, sem_ga)
        pltpu.async_copy(feat_hbm.at[slot0.at[2]], rows_b, sem_gb)

        plsc.subcore_barrier()

        def scale(rows, wslot, wrow):
            def scale_body(g, carry):
                wv = wslot[wrow, pl.ds(g * 16, 16)]
                for e in range(16):
                    w = wv[e]
                    r0 = g * 16 + e
                    for j in range(D // 16):
                        rows[r0, pl.ds(j * 16, 16)] = (
                            rows[r0, pl.ds(j * 16, 16)] * w)
                return carry
            lax.fori_loop(0, _K // 16, scale_body, 0)

        def body(q, carry):
            for half in range(2):
                slot, wslot = slots[half], wslots[half]
                sem_e, sem_w = sems_e[half], sems_w[half]
                other = slots[1 - half]
                sem_eo = sems_e[1 - half]
                p = 2 * q + half
                pltpu.make_async_copy(feat_hbm.at[slot.at[0]], rows_a,
                                      sem_ga).wait()
                pltpu.make_async_copy(wdata_pair(p), wslot, sem_w).wait()
                scale(rows_a, wslot, 0)
                pltpu.async_copy(rows_a, agg.at[slot.at[1]], sem_sa, add=True)
                pltpu.make_async_copy(feat_hbm.at[slot.at[2]], rows_b,
                                      sem_gb).wait()
                scale(rows_b, wslot, 1)
                pltpu.async_copy(rows_b, agg.at[slot.at[3]], sem_sb, add=True)
                @pl.when(p < my_npair - 1)
                def _advance():
                    pltpu.make_async_copy(edata_pair(p + 1), other,
                                          sem_eo).wait()
                    pltpu.make_async_copy(rows_a, agg.at[slot.at[1]],
                                          sem_sa).wait()
                    pltpu.async_copy(feat_hbm.at[other.at[0]], rows_a, sem_ga)
                    pltpu.make_async_copy(rows_b, agg.at[slot.at[3]],
                                          sem_sb).wait()
                    pltpu.async_copy(feat_hbm.at[other.at[2]], rows_b, sem_gb)

                    @pl.when(p < my_npair - 2)
                    def _prefetch():
                        pltpu.async_copy(edata_pair(p + 2), slot, sem_e)
                        pltpu.async_copy(wdata_pair(p + 2), wslot, sem_w)

                @pl.when(p == my_npair - 1)
                def _drain():
                    pltpu.make_async_copy(rows_a, agg.at[slot.at[1]],
                                          sem_sa).wait()
                    pltpu.make_async_copy(rows_b, agg.at[slot.at[3]],
                                          sem_sb).wait()
            return carry

        lax.fori_loop(0, my_npair // 2, body, 0)

        plsc.subcore_barrier()
        pltpu.sync_copy(agg.at[pl.ds(base_row, ROWS_PER_TILE)],
                        out_hbm.at[cid, pl.ds(base_row, ROWS_PER_TILE)])

        @pl.when(sid == _NS - 1)
        def _copy_rem():
            pltpu.sync_copy(
                agg.at[pl.ds(_NS * ROWS_PER_TILE, REM_ROWS)],
                out_hbm.at[cid, pl.ds(_NS * ROWS_PER_TILE, REM_ROWS)])

    return sc_scatter


def _tc_body(x_ref, bp_ref, k_ref, bias_ref, skip_ref, o_ref):
    kmat = k_ref[...]
    k2 = kmat * skip_ref[...]
    bsum = bp_ref[0] + bp_ref[1]
    acc = jnp.dot(x_ref[...], k2, preferred_element_type=jnp.float32,
                  precision=lax.Precision.HIGHEST)
    acc = acc + jnp.dot(bsum, kmat, preferred_element_type=jnp.float32,
                        precision=lax.Precision.HIGHEST)
    acc = acc + bias_ref[...]
    pos = acc > 0.0
    safe = jnp.where(pos, 0.0, acc)
    o_ref[...] = jnp.where(
        pos, _SELU_SCALE * acc,
        (_SELU_SCALE * _SELU_ALPHA) * (jnp.exp(safe) - 1.0))


def _tc_fused(features, bp, kmat, bias, skip):
    N, D = features.shape
    C = kmat.shape[1]
    BM = 1000
    grid = (N // BM,)
    return pl.pallas_call(
        _tc_body,
        grid=grid,
        in_specs=[
            pl.BlockSpec((BM, D), lambda i: (i, 0)),
            pl.BlockSpec((2, BM, C), lambda i: (0, i, 0)),
            pl.BlockSpec((D, C), lambda i: (0, 0)),
            pl.BlockSpec((1, C), lambda i: (0, 0)),
            pl.BlockSpec((1, C), lambda i: (0, 0)),
        ],
        out_specs=pl.BlockSpec((BM, C), lambda i: (i, 0)),
        out_shape=jax.ShapeDtypeStruct((N, C), jnp.float32),
    )(features, bp, kmat, bias, skip)


def kernel(features, edge_index, edge_weight, kernel, bias, skip_weight):
    N, D = features.shape
    C = kernel.shape[1]
    E = edge_weight.shape[0]
    C0, C1 = _chunk_split(E)
    TCH = _NS * (C0 + C1)
    E2 = TCH * _K
    dst = edge_index[0]
    src = edge_index[1]
    pad = E2 - E
    src_p = jnp.concatenate(
        [src, jnp.zeros((pad,), jnp.int32)]).reshape(TCH // 2, 2, _K)
    dst_p = jnp.concatenate(
        [dst, jnp.zeros((pad,), jnp.int32)]).reshape(TCH // 2, 2, _K)
    w_p = jnp.concatenate(
        [edge_weight, jnp.zeros((pad,), jnp.float32)]
    ).reshape(TCH // 2, 2, _K)
    eidx = jnp.pad(
        jnp.stack([src_p[:, 0], dst_p[:, 0],
                   src_p[:, 1], dst_p[:, 1]], axis=1),
        ((0, 0), (0, 4), (0, 0)))
    zrows = (N // _NS) // 8 * 8
    zeros = jnp.zeros((zrows, D), jnp.float32)
    bp = _make_sc_scatter(N, D, E)(eidx, w_p, features, zeros)
    return _tc_fused(features, bp, kernel,
                     bias.reshape(1, C), skip_weight.reshape(1, C))
